# Initial kernel scaffold; baseline (speedup 1.0000x reference)
import jax
import jax.numpy as jnp
from jax.experimental import pallas as pl


def kernel(h, batch, W1, b1, W2, b2, W3, b3):
    raise NotImplementedError("write your pallas kernel here")


# speedup vs baseline: 6.0260x; 6.0260x over previous
import functools

import jax
import jax.numpy as jnp
from jax.experimental import pallas as pl
from jax.experimental.pallas import tpu as pltpu

ROWS = 320000
D_IN = 128
D_H = 256
D_M = 64
NUM_SEGMENTS = 512
BLK = 3200
NBLK = ROWS // BLK
S_TILE = 32


def _body(h_ref, ids_ref, w1_ref, b1_ref, w2_ref, b2_ref, w3_ref, b3_ref,
          out_ref, acc_ref):
    i = pl.program_id(0)

    @pl.when(i == 0)
    def _init():
        acc_ref[...] = jnp.zeros_like(acc_ref)

    act = jax.nn.softplus(
        jnp.dot(h_ref[...], w1_ref[...], preferred_element_type=jnp.float32)
        + b1_ref[...])

    ids_row = ids_ref[0]
    s_lo = jnp.min(ids_row)
    s_hi = jnp.max(ids_row)
    base0 = (s_lo // S_TILE) * S_TILE
    ntiles = (s_hi - base0) // S_TILE + 1

    def tile_body(t, _):
        base = base0 + t * S_TILE
        iota = jax.lax.broadcasted_iota(jnp.int32, (S_TILE, BLK), 0) + base
        oh = (iota == ids_row).astype(jnp.float32)
        partial = jnp.dot(oh, act, preferred_element_type=jnp.float32)
        acc_ref[pl.ds(base, S_TILE), :] += partial
        return 0

    jax.lax.fori_loop(0, ntiles, tile_body, 0)

    @pl.when(i == NBLK - 1)
    def _tail():
        x = jax.nn.softplus(acc_ref[...])
        x = jnp.dot(x, w2_ref[...], preferred_element_type=jnp.float32) + b2_ref[...]
        x = jax.nn.softplus(x)
        out_ref[...] = jnp.sum(x * w3_ref[...], axis=1, keepdims=True) + b3_ref[0, 0]


@functools.partial(jax.jit, static_argnames=())
def _run(h, ids3, W1, b1, W2, b2, w3row, b3):
    return pl.pallas_call(
        _body,
        grid=(NBLK,),
        in_specs=[
            pl.BlockSpec((BLK, D_IN), lambda i: (i, 0)),
            pl.BlockSpec((1, 1, BLK), lambda i: (i, 0, 0)),
            pl.BlockSpec((D_IN, D_H), lambda i: (0, 0)),
            pl.BlockSpec((1, D_H), lambda i: (0, 0)),
            pl.BlockSpec((D_H, D_M), lambda i: (0, 0)),
            pl.BlockSpec((1, D_M), lambda i: (0, 0)),
            pl.BlockSpec((1, D_M), lambda i: (0, 0)),
            pl.BlockSpec((1, 1), lambda i: (0, 0)),
        ],
        out_specs=pl.BlockSpec((NUM_SEGMENTS, 1), lambda i: (0, 0)),
        out_shape=jax.ShapeDtypeStruct((NUM_SEGMENTS, 1), jnp.float32),
        scratch_shapes=[pltpu.VMEM((NUM_SEGMENTS, D_H), jnp.float32)],
    )(h, ids3, W1, b1, W2, b2, w3row, b3)


def kernel(h, batch, W1, b1, W2, b2, W3, b3):
    ids3 = batch.astype(jnp.int32).reshape(NBLK, 1, BLK)
    return _run(h, ids3, W1, b1.reshape(1, D_H), W2, b2.reshape(1, D_M),
                W3.reshape(1, D_M), b3.reshape(1, 1))


# speedup vs baseline: 6.0935x; 1.0112x over previous
import functools

import jax
import jax.numpy as jnp
from jax.experimental import pallas as pl
from jax.experimental.pallas import tpu as pltpu

ROWS = 320000
D_IN = 128
D_H = 256
D_M = 64
NUM_SEGMENTS = 512
BLK = 3200
NBLK = ROWS // BLK
S_TILE = 16


def _body(h_ref, ids_ref, w1_ref, b1_ref, w2_ref, b2_ref, w3_ref, b3_ref,
          out_ref, acc_ref):
    i = pl.program_id(0)

    @pl.when(i == 0)
    def _init():
        acc_ref[...] = jnp.zeros_like(acc_ref)

    act = jax.nn.softplus(
        jnp.dot(h_ref[...], w1_ref[...], preferred_element_type=jnp.float32)
        + b1_ref[...])

    ids_row = ids_ref[0]
    s_lo = jnp.min(ids_row)
    s_hi = jnp.max(ids_row)
    base0 = (s_lo // S_TILE) * S_TILE
    ntiles = (s_hi - base0) // S_TILE + 1

    def tile_body(t, _):
        base = base0 + t * S_TILE
        iota = jax.lax.broadcasted_iota(jnp.int32, (S_TILE, BLK), 0) + base
        oh = (iota == ids_row).astype(jnp.float32)
        partial = jnp.dot(oh, act, preferred_element_type=jnp.float32)
        acc_ref[pl.ds(base, S_TILE), :] += partial
        return 0

    jax.lax.fori_loop(0, ntiles, tile_body, 0)

    @pl.when(i == NBLK - 1)
    def _tail():
        x = jax.nn.softplus(acc_ref[...])
        x = jnp.dot(x, w2_ref[...], preferred_element_type=jnp.float32) + b2_ref[...]
        x = jax.nn.softplus(x)
        out_ref[...] = jnp.sum(x * w3_ref[...], axis=1, keepdims=True) + b3_ref[0, 0]


@functools.partial(jax.jit, static_argnames=())
def _run(h, ids3, W1, b1, W2, b2, w3row, b3):
    return pl.pallas_call(
        _body,
        grid=(NBLK,),
        in_specs=[
            pl.BlockSpec((BLK, D_IN), lambda i: (i, 0)),
            pl.BlockSpec((1, 1, BLK), lambda i: (i, 0, 0)),
            pl.BlockSpec((D_IN, D_H), lambda i: (0, 0)),
            pl.BlockSpec((1, D_H), lambda i: (0, 0)),
            pl.BlockSpec((D_H, D_M), lambda i: (0, 0)),
            pl.BlockSpec((1, D_M), lambda i: (0, 0)),
            pl.BlockSpec((1, D_M), lambda i: (0, 0)),
            pl.BlockSpec((1, 1), lambda i: (0, 0)),
        ],
        out_specs=pl.BlockSpec((NUM_SEGMENTS, 1), lambda i: (0, 0)),
        out_shape=jax.ShapeDtypeStruct((NUM_SEGMENTS, 1), jnp.float32),
        scratch_shapes=[pltpu.VMEM((NUM_SEGMENTS, D_H), jnp.float32)],
    )(h, ids3, W1, b1, W2, b2, w3row, b3)


def kernel(h, batch, W1, b1, W2, b2, W3, b3):
    ids3 = batch.astype(jnp.int32).reshape(NBLK, 1, BLK)
    return _run(h, ids3, W1, b1.reshape(1, D_H), W2, b2.reshape(1, D_M),
                W3.reshape(1, D_M), b3.reshape(1, 1))


# speedup vs baseline: 9.6565x; 1.5847x over previous
import functools

import jax
import jax.numpy as jnp
from jax.experimental import pallas as pl
from jax.experimental.pallas import tpu as pltpu

ROWS = 320000
D_IN = 128
D_H = 256
D_M = 64
NUM_SEGMENTS = 512
BLK = 3200
NBLK = ROWS // BLK
S_TILE = 16


def _body(h_ref, ids_ref, w1_ref, b1_ref, w2_ref, b2_ref, w3_ref, b3_ref,
          out_ref, acc_ref):
    i = pl.program_id(0)

    @pl.when(i == 0)
    def _init():
        acc_ref[...] = jnp.zeros_like(acc_ref)

    y = jnp.dot(h_ref[...], w1_ref[...], preferred_element_type=jnp.float32) + b1_ref[...]
    act = jnp.log2(1.0 + jnp.exp2(y))

    ids_row = ids_ref[0]
    s_lo = jnp.min(ids_row)
    s_hi = jnp.max(ids_row)
    base0 = (s_lo // S_TILE) * S_TILE
    ntiles = (s_hi - base0) // S_TILE + 1

    def tile_body(t, _):
        base = base0 + t * S_TILE
        iota = jax.lax.broadcasted_iota(jnp.int32, (S_TILE, BLK), 0) + base
        oh = (iota == ids_row).astype(jnp.float32)
        partial = jnp.dot(oh, act, preferred_element_type=jnp.float32)
        acc_ref[pl.ds(base, S_TILE), :] += partial
        return 0

    jax.lax.fori_loop(0, ntiles, tile_body, 0)

    @pl.when(i == NBLK - 1)
    def _tail():
        x = jax.nn.softplus(acc_ref[...] * jnp.float32(0.6931471805599453))
        x = jnp.dot(x, w2_ref[...], preferred_element_type=jnp.float32) + b2_ref[...]
        x = jax.nn.softplus(x)
        out_ref[...] = jnp.sum(x * w3_ref[...], axis=1, keepdims=True) + b3_ref[0, 0]


@functools.partial(jax.jit, static_argnames=())
def _run(h, ids3, W1, b1, W2, b2, w3row, b3):
    return pl.pallas_call(
        _body,
        grid=(NBLK,),
        in_specs=[
            pl.BlockSpec((BLK, D_IN), lambda i: (i, 0)),
            pl.BlockSpec((1, 1, BLK), lambda i: (i, 0, 0)),
            pl.BlockSpec((D_IN, D_H), lambda i: (0, 0)),
            pl.BlockSpec((1, D_H), lambda i: (0, 0)),
            pl.BlockSpec((D_H, D_M), lambda i: (0, 0)),
            pl.BlockSpec((1, D_M), lambda i: (0, 0)),
            pl.BlockSpec((1, D_M), lambda i: (0, 0)),
            pl.BlockSpec((1, 1), lambda i: (0, 0)),
        ],
        out_specs=pl.BlockSpec((NUM_SEGMENTS, 1), lambda i: (0, 0)),
        out_shape=jax.ShapeDtypeStruct((NUM_SEGMENTS, 1), jnp.float32),
        scratch_shapes=[pltpu.VMEM((NUM_SEGMENTS, D_H), jnp.float32)],
    )(h, ids3, W1, b1, W2, b2, w3row, b3)


def kernel(h, batch, W1, b1, W2, b2, W3, b3):
    ids3 = batch.astype(jnp.int32).reshape(NBLK, 1, BLK)
    log2e = jnp.float32(1.4426950408889634)
    return _run(h, ids3, W1 * log2e, (b1 * log2e).reshape(1, D_H), W2,
                b2.reshape(1, D_M), W3.reshape(1, D_M), b3.reshape(1, 1))
